# baseline (speedup 1.0000x reference)
import jax
import jax.numpy as jnp
from jax import lax
from jax.experimental import pallas as pl
from jax.experimental.pallas import tpu as pltpu
from jax.experimental.pallas import tpu_sc as plsc

VOCAB = 100000
EMBED = 64
NTOK = 200
BATCH = 4096

NC = 2
NS = 16
NW = NC * NS
SEQ_PER_W = BATCH // NW
SPLIT = (96, 104)
LANES = 16
NBUF = 3


def _body(tokens_hbm, table_hbm, pos_hbm, out_hbm, idx_v, rows_v,
          pos_v, g0, g1, g2, w0, w1, w2):
    sem_g = (g0, g1, g2)
    sem_w = (w0, w1, w2)
    wid = lax.axis_index("s") * NC + lax.axis_index("c")
    pltpu.sync_copy(tokens_hbm.at[pl.ds(wid * SEQ_PER_W, SEQ_PER_W)], idx_v)
    pltpu.sync_copy(pos_hbm, pos_v)
    obase = wid * SEQ_PER_W

    def start_gather(s, b):
        pltpu.async_copy(table_hbm.at[idx_v.at[s, pl.ds(0, SPLIT[0])]],
                         rows_v.at[b, pl.ds(0, SPLIT[0])], sem_g[b])
        pltpu.async_copy(table_hbm.at[idx_v.at[s, pl.ds(SPLIT[0], SPLIT[1])]],
                         rows_v.at[b, pl.ds(SPLIT[0], SPLIT[1])], sem_g[b])

    def wait_gather(s, b):
        pltpu.make_async_copy(table_hbm.at[idx_v.at[s, pl.ds(0, SPLIT[0])]],
                              rows_v.at[b, pl.ds(0, SPLIT[0])], sem_g[b]).wait()
        pltpu.make_async_copy(table_hbm.at[idx_v.at[s, pl.ds(SPLIT[0], SPLIT[1])]],
                              rows_v.at[b, pl.ds(SPLIT[0], SPLIT[1])], sem_g[b]).wait()

    def start_write(s, b):
        pltpu.async_copy(rows_v.at[b], out_hbm.at[obase + s], sem_w[b])

    def wait_write(s, b):
        pltpu.make_async_copy(rows_v.at[b], out_hbm.at[obase + s],
                              sem_w[b]).wait()

    def add_pos(b):
        def radd(r, c2):
            for c in range(EMBED // LANES):
                ds = pl.ds(c * LANES, LANES)
                rows_v[b, r, ds] = rows_v[b, r, ds] + pos_v[r, ds]
            return c2

        lax.fori_loop(0, NTOK, radd, 0, unroll=8)

    def seq_body(s, b, prefetch, reclaim):
        pb = (b + 2) % NBUF
        if prefetch:
            if reclaim:
                wait_write(s - 1, pb)
            start_gather(s + 2, pb)
        wait_gather(s, b)
        add_pos(b)
        start_write(s, b)

    start_gather(0, 0)
    start_gather(1, 1)
    seq_body(0, 0, prefetch=True, reclaim=False)

    def outer(k, carry):
        s0 = 1 + 3 * k
        for j, b in enumerate((1, 2, 0)):
            seq_body(s0 + j, b, prefetch=True, reclaim=True)
        return carry

    lax.fori_loop(0, 41, outer, 0)
    seq_body(124, 1, prefetch=True, reclaim=True)
    seq_body(125, 2, prefetch=True, reclaim=True)
    seq_body(126, 0, prefetch=False, reclaim=False)
    seq_body(127, 1, prefetch=False, reclaim=False)
    wait_write(125, 2)
    wait_write(126, 0)
    wait_write(127, 1)


def kernel(tokens, token_embedding, positional_embedding):
    grid_kernel = pl.kernel(
        _body,
        out_type=jax.ShapeDtypeStruct((BATCH, NTOK, EMBED), jnp.float32),
        mesh=plsc.VectorSubcoreMesh(core_axis_name="c", subcore_axis_name="s"),
        compiler_params=pltpu.CompilerParams(use_tc_tiling_on_sc=False),
        scratch_types=[
            pltpu.VMEM((SEQ_PER_W, NTOK), jnp.int32),
            pltpu.VMEM((NBUF, NTOK, EMBED), jnp.float32),
            pltpu.VMEM((NTOK, EMBED), jnp.float32),
            pltpu.SemaphoreType.DMA,
            pltpu.SemaphoreType.DMA,
            pltpu.SemaphoreType.DMA,
            pltpu.SemaphoreType.DMA,
            pltpu.SemaphoreType.DMA,
            pltpu.SemaphoreType.DMA,
        ],
    )
    return grid_kernel(tokens.astype(jnp.int32), token_embedding,
                       positional_embedding)


# speedup vs baseline: 1.1581x; 1.1581x over previous
import jax
import jax.numpy as jnp
from jax import lax
from jax.experimental import pallas as pl
from jax.experimental.pallas import tpu as pltpu
from jax.experimental.pallas import tpu_sc as plsc

VOCAB = 100000
EMBED = 64
NTOK = 200
BATCH = 4096

NC = 2
NS = 16
NW = NC * NS
SEQ_PER_W = BATCH // NW
HALF = NTOK // 2
NCHUNK = SEQ_PER_W * 2
LANES = 16
NBUF = 4


def _body(tokens_hbm, table_hbm, pos_hbm, out_hbm, idx_v, rows_v,
          pos_v, g0, g1, g2, g3, w0, w1, w2, w3):
    sem_g = (g0, g1, g2, g3)
    sem_w = (w0, w1, w2, w3)
    wid = lax.axis_index("s") * NC + lax.axis_index("c")
    pltpu.sync_copy(tokens_hbm.at[wid], idx_v)
    pltpu.sync_copy(pos_hbm, pos_v)
    obase = wid * SEQ_PER_W

    def start_gather(s, b):
        pltpu.async_copy(table_hbm.at[idx_v.at[2 * s]], rows_v.at[b, 0], sem_g[b])
        pltpu.async_copy(table_hbm.at[idx_v.at[2 * s + 1]], rows_v.at[b, 1], sem_g[b])

    def wait_gather(s, b):
        pltpu.make_async_copy(table_hbm.at[idx_v.at[2 * s]],
                              rows_v.at[b, 0], sem_g[b]).wait()
        pltpu.make_async_copy(table_hbm.at[idx_v.at[2 * s + 1]],
                              rows_v.at[b, 1], sem_g[b]).wait()

    def start_write(s, b):
        for h in range(2):
            pltpu.async_copy(rows_v.at[b, h],
                             out_hbm.at[obase + s, pl.ds(h * HALF, HALF)],
                             sem_w[b])

    def wait_write(s, b):
        for h in range(2):
            pltpu.make_async_copy(rows_v.at[b, h],
                                  out_hbm.at[obase + s, pl.ds(h * HALF, HALF)],
                                  sem_w[b]).wait()

    def add_pos(b):
        def radd(r, c2):
            for h in range(2):
                for c in range(EMBED // LANES):
                    ds = pl.ds(c * LANES, LANES)
                    rows_v[b, h, r, ds] = rows_v[b, h, r, ds] + pos_v[h, r, ds]
            return c2

        lax.fori_loop(0, HALF, radd, 0, unroll=8)

    def seq_body(s, b, prefetch, reclaim):
        pb = (b + NBUF - 1) % NBUF
        if prefetch:
            if reclaim:
                wait_write(s - 1, pb)
            start_gather(s + NBUF - 1, pb)
        wait_gather(s, b)
        add_pos(b)
        start_write(s, b)

    for s in range(NBUF - 1):
        start_gather(s, s)
    seq_body(0, 0, prefetch=True, reclaim=False)

    def outer(k, carry):
        s0 = 1 + NBUF * k
        for j in range(NBUF):
            seq_body(s0 + j, (1 + j) % NBUF, prefetch=True, reclaim=True)
        return carry

    lax.fori_loop(0, (SEQ_PER_W - NBUF) // NBUF, outer, 0)
    for s in range(SEQ_PER_W - NBUF + 1, SEQ_PER_W):
        seq_body(s, s % NBUF, prefetch=False, reclaim=False)
    for s in range(SEQ_PER_W - NBUF, SEQ_PER_W):
        wait_write(s, s % NBUF)


def kernel(tokens, token_embedding, positional_embedding):
    tokens_r = tokens.reshape(NW, NCHUNK, HALF).astype(jnp.int32)
    pos_r = positional_embedding.reshape(2, HALF, EMBED)
    grid_kernel = pl.kernel(
        _body,
        out_type=jax.ShapeDtypeStruct((BATCH, NTOK, EMBED), jnp.float32),
        mesh=plsc.VectorSubcoreMesh(core_axis_name="c", subcore_axis_name="s"),
        compiler_params=pltpu.CompilerParams(use_tc_tiling_on_sc=False),
        scratch_types=[
            pltpu.VMEM((NCHUNK, HALF), jnp.int32),
            pltpu.VMEM((NBUF, 2, HALF, EMBED), jnp.float32),
            pltpu.VMEM((2, HALF, EMBED), jnp.float32),
        ] + [pltpu.SemaphoreType.DMA] * (2 * NBUF),
    )
    return grid_kernel(tokens_r, token_embedding, pos_r)
